# Initial kernel scaffold; baseline (speedup 1.0000x reference)
import jax
import jax.numpy as jnp
from jax.experimental import pallas as pl


def kernel(x, edge_index, batch, W1, b1, W2, b2, gateW, gateb, f1W, f1b, f2W, f2b, d1W, d1b, d2W, d2b):
    raise NotImplementedError("write your pallas kernel here")


# speedup vs baseline: 4.1958x; 4.1958x over previous
import functools

import jax
import jax.numpy as jnp
from jax import lax
from jax.experimental import pallas as pl
from jax.experimental.pallas import tpu as pltpu
from jax.experimental.pallas import tpu_sc as plsc

N = 10000
E = 160000
NC = 2
NS = 16
G = 128
R = 1280
EP = R * G
NP = N + 8

_mesh = lambda: plsc.VectorSubcoreMesh(core_axis_name="c", subcore_axis_name="s")

_DEG_R_SUB = R // (NC * NS)


def _deg_body(dst_hbm, zeros_hbm, ones_hbm, dega_hbm, degb_hbm,
              idx_v, ones_v, acc_sh):
    c = lax.axis_index("c")
    s = lax.axis_index("s")
    w = c * NS + s

    @pl.when(s == 0)
    def _():
        pltpu.sync_copy(zeros_hbm, acc_sh)
    pltpu.sync_copy(ones_hbm, ones_v)
    pltpu.sync_copy(dst_hbm.at[pl.ds(w * _DEG_R_SUB, _DEG_R_SUB)], idx_v)
    plsc.subcore_barrier()

    def body(j, _):
        pltpu.sync_copy(ones_v, acc_sh.at[idx_v.at[j]], add=True)
        return 0

    lax.fori_loop(0, _DEG_R_SUB, body, 0)
    plsc.subcore_barrier()

    @pl.when(jnp.logical_and(c == 0, s == 0))
    def _():
        pltpu.sync_copy(acc_sh, dega_hbm)

    @pl.when(jnp.logical_and(c == 1, s == 0))
    def _():
        pltpu.sync_copy(acc_sh, degb_hbm)


def _deg_kernel(dstp, zeros128, ones128):
    f = pl.kernel(
        _deg_body,
        out_type=[jax.ShapeDtypeStruct((NP, 128), jnp.float32),
                  jax.ShapeDtypeStruct((NP, 128), jnp.float32)],
        mesh=_mesh(),
        scratch_types=[pltpu.VMEM((_DEG_R_SUB, G), jnp.int32),
                       pltpu.VMEM((G, 128), jnp.float32),
                       pltpu.VMEM_SHARED((NP, 128), jnp.float32)],
    )
    return f(dstp, zeros128, ones128)


_SP1_R_SUB = R // NS
_ROWS_PER_SUB = N // 10


def _spmm128_body(hs_a, hs_b, src_hbm, dst_hbm, outa_hbm, outb_hbm,
                  src_v, dst_v, rows_v, acc_sh, sem):
    c = lax.axis_index("c")
    s = lax.axis_index("s")

    def run(hs, out_hbm):
        @pl.when(s < 10)
        def _():
            r = pl.ds(s * _ROWS_PER_SUB, _ROWS_PER_SUB)
            pltpu.sync_copy(hs.at[r], acc_sh.at[r])
        pltpu.sync_copy(src_hbm.at[pl.ds(s * _SP1_R_SUB, _SP1_R_SUB)], src_v)
        pltpu.sync_copy(dst_hbm.at[pl.ds(s * _SP1_R_SUB, _SP1_R_SUB)], dst_v)
        plsc.subcore_barrier()

        def body_j(j, _):
            pltpu.async_copy(hs.at[src_v.at[j]], rows_v, sem).wait()
            pltpu.sync_copy(rows_v, acc_sh.at[dst_v.at[j]], add=True)
            return 0

        lax.fori_loop(0, _SP1_R_SUB, body_j, 0)
        plsc.subcore_barrier()

        @pl.when(s < 10)
        def _():
            r = pl.ds(s * _ROWS_PER_SUB, _ROWS_PER_SUB)
            pltpu.sync_copy(acc_sh.at[r], out_hbm.at[r])

    @pl.when(c == 0)
    def _():
        run(hs_a, outa_hbm)

    @pl.when(c == 1)
    def _():
        run(hs_b, outb_hbm)


def _spmm128(hs_a, hs_b, srcp, dstp):
    f = pl.kernel(
        _spmm128_body,
        out_type=[jax.ShapeDtypeStruct((NP, 128), jnp.float32),
                  jax.ShapeDtypeStruct((NP, 128), jnp.float32)],
        mesh=_mesh(),
        scratch_types=[pltpu.VMEM((_SP1_R_SUB, G), jnp.int32),
                       pltpu.VMEM((_SP1_R_SUB, G), jnp.int32),
                       pltpu.VMEM((G, 128), jnp.float32),
                       pltpu.VMEM_SHARED((NP, 128), jnp.float32),
                       pltpu.SemaphoreType.DMA],
    )
    return f(hs_a, hs_b, srcp, dstp)


_SP2_R_SUB = R // (NC * NS)


def _spmm_pad_body(hs, zeros_hbm, src_hbm, dst_hbm, outa_hbm, outb_hbm,
                   src_v, dst_v, rows_v, acc_sh, sem):
    c = lax.axis_index("c")
    s = lax.axis_index("s")
    w = c * NS + s

    @pl.when(jnp.logical_and(c == 0, s < 10))
    def _():
        r = pl.ds(s * _ROWS_PER_SUB, _ROWS_PER_SUB)
        pltpu.sync_copy(hs.at[r], acc_sh.at[r])

    @pl.when(jnp.logical_and(c == 1, s < 10))
    def _():
        r = pl.ds(s * _ROWS_PER_SUB, _ROWS_PER_SUB)
        pltpu.sync_copy(zeros_hbm.at[r], acc_sh.at[r])

    pltpu.sync_copy(src_hbm.at[pl.ds(w * _SP2_R_SUB, _SP2_R_SUB)], src_v)
    pltpu.sync_copy(dst_hbm.at[pl.ds(w * _SP2_R_SUB, _SP2_R_SUB)], dst_v)
    plsc.subcore_barrier()

    def body_j(j, _):
        pltpu.async_copy(hs.at[src_v.at[j]], rows_v, sem).wait()
        pltpu.sync_copy(rows_v, acc_sh.at[dst_v.at[j]], add=True)
        return 0

    lax.fori_loop(0, _SP2_R_SUB, body_j, 0)
    plsc.subcore_barrier()

    @pl.when(jnp.logical_and(c == 0, s < 10))
    def _():
        r = pl.ds(s * _ROWS_PER_SUB, _ROWS_PER_SUB)
        pltpu.sync_copy(acc_sh.at[r], outa_hbm.at[r])

    @pl.when(jnp.logical_and(c == 1, s < 10))
    def _():
        r = pl.ds(s * _ROWS_PER_SUB, _ROWS_PER_SUB)
        pltpu.sync_copy(acc_sh.at[r], outb_hbm.at[r])


def _spmm_pad(hs, zerosNP, srcp, dstp):
    f = pl.kernel(
        _spmm_pad_body,
        out_type=[jax.ShapeDtypeStruct((NP, 128), jnp.float32),
                  jax.ShapeDtypeStruct((NP, 128), jnp.float32)],
        mesh=_mesh(),
        scratch_types=[pltpu.VMEM((_SP2_R_SUB, G), jnp.int32),
                       pltpu.VMEM((_SP2_R_SUB, G), jnp.int32),
                       pltpu.VMEM((G, 128), jnp.float32),
                       pltpu.VMEM_SHARED((NP, 128), jnp.float32),
                       pltpu.SemaphoreType.DMA],
    )
    return f(hs, zerosNP, srcp, dstp)


_GAT_R_SUB = R // (NC * NS)


def _gather_body(u_hbm, v_hbm, src_hbm, dst_hbm, ug_hbm, vg_hbm,
                 idx_v, rows_v, sem):
    c = lax.axis_index("c")
    s = lax.axis_index("s")
    w = c * NS + s

    def run(tab_hbm, ind_hbm, out_hbm):
        pltpu.sync_copy(ind_hbm.at[pl.ds(w * _GAT_R_SUB, _GAT_R_SUB)], idx_v)

        def body_j(j, _):
            pltpu.async_copy(tab_hbm.at[idx_v.at[j]], rows_v, sem).wait()
            pltpu.sync_copy(rows_v,
                            out_hbm.at[pl.ds((w * _GAT_R_SUB + j) * G, G)])
            return 0

        lax.fori_loop(0, _GAT_R_SUB, body_j, 0)

    run(u_hbm, src_hbm, ug_hbm)
    run(v_hbm, dst_hbm, vg_hbm)


def _decode_gather(u, v, srcp, dstp):
    f = pl.kernel(
        _gather_body,
        out_type=[jax.ShapeDtypeStruct((EP, 128), jnp.float32),
                  jax.ShapeDtypeStruct((EP, 128), jnp.float32)],
        mesh=_mesh(),
        scratch_types=[pltpu.VMEM((_GAT_R_SUB, G), jnp.int32),
                       pltpu.VMEM((G, 128), jnp.float32),
                       pltpu.SemaphoreType.DMA],
    )
    return f(u, v, srcp, dstp)


_ROWS_BLK = 1000


def _mm1_body(x_ref, w_ref, dega_ref, degb_ref, hsa_ref, hsb_ref, dinv_ref):
    deg = dega_ref[:, 0:1] + degb_ref[:, 0:1] + 1.0
    dinv = lax.rsqrt(deg)
    hs = jnp.dot(x_ref[...], w_ref[...], preferred_element_type=jnp.float32)
    hs = hs * dinv
    hsa_ref[...] = hs[:, :128]
    hsb_ref[...] = hs[:, 128:]
    dinv_ref[...] = dinv


def _mm1(x, W1, dega, degb):
    return pl.pallas_call(
        _mm1_body,
        grid=(N // _ROWS_BLK,),
        in_specs=[
            pl.BlockSpec((_ROWS_BLK, 256), lambda i: (i, 0)),
            pl.BlockSpec((256, 256), lambda i: (0, 0)),
            pl.BlockSpec((_ROWS_BLK, 128), lambda i: (i, 0)),
            pl.BlockSpec((_ROWS_BLK, 128), lambda i: (i, 0)),
        ],
        out_specs=[
            pl.BlockSpec((_ROWS_BLK, 128), lambda i: (i, 0)),
            pl.BlockSpec((_ROWS_BLK, 128), lambda i: (i, 0)),
            pl.BlockSpec((_ROWS_BLK, 1), lambda i: (i, 0)),
        ],
        out_shape=[jax.ShapeDtypeStruct((NP, 128), jnp.float32),
                   jax.ShapeDtypeStruct((NP, 128), jnp.float32),
                   jax.ShapeDtypeStruct((N, 1), jnp.float32)],
    )(x, W1, dega, degb)


def _mm2_body(a_ref, b_ref, dinv_ref, b1a_ref, b1b_ref, w2a_ref, w2b_ref,
              hs2_ref):
    dinv = dinv_ref[...]
    ha = jnp.maximum(a_ref[...] * dinv + b1a_ref[...], 0.0)
    hb = jnp.maximum(b_ref[...] * dinv + b1b_ref[...], 0.0)
    zl = (jnp.dot(ha, w2a_ref[...], preferred_element_type=jnp.float32)
          + jnp.dot(hb, w2b_ref[...], preferred_element_type=jnp.float32))
    hs2 = zl * dinv
    hs2_ref[...] = jnp.pad(hs2, ((0, 0), (0, 64)))


def _mm2(agg1a, agg1b, dinv, b1a, b1b, W2a, W2b):
    return pl.pallas_call(
        _mm2_body,
        grid=(N // _ROWS_BLK,),
        in_specs=[
            pl.BlockSpec((_ROWS_BLK, 128), lambda i: (i, 0)),
            pl.BlockSpec((_ROWS_BLK, 128), lambda i: (i, 0)),
            pl.BlockSpec((_ROWS_BLK, 1), lambda i: (i, 0)),
            pl.BlockSpec((1, 128), lambda i: (0, 0)),
            pl.BlockSpec((1, 128), lambda i: (0, 0)),
            pl.BlockSpec((128, 64), lambda i: (0, 0)),
            pl.BlockSpec((128, 64), lambda i: (0, 0)),
        ],
        out_specs=pl.BlockSpec((_ROWS_BLK, 128), lambda i: (i, 0)),
        out_shape=jax.ShapeDtypeStruct((NP, 128), jnp.float32),
    )(agg1a, agg1b, dinv, b1a, b1b, W2a, W2b)


def _pool_body(a_ref, b_ref, dinv_ref, b2_ref, gw_ref, gb_ref,
               f1w_ref, f1b_ref, f2w_ref, f2b_ref, d1wa_ref, d1wb_ref,
               d1b_ref, u_ref, v_ref):
    z = a_ref[0:N, 0:64] + b_ref[0:N, 0:64]
    z = z * dinv_ref[...] + b2_ref[...]
    gate = jnp.dot(z, gw_ref[...], preferred_element_type=jnp.float32) + gb_ref[...]
    gmax = jnp.max(gate, axis=0, keepdims=True)
    e = jnp.exp(gate - gmax)
    den = jnp.sum(e, axis=0, keepdims=True)
    attn = e / den
    g = jnp.sum(attn * z, axis=0, keepdims=True)
    f1 = jnp.maximum(
        jnp.dot(g, f1w_ref[...], preferred_element_type=jnp.float32)
        + f1b_ref[...], 0.0)
    film = jnp.dot(f1, f2w_ref[...], preferred_element_type=jnp.float32) + f2b_ref[...]
    gamma = film[:, :64]
    beta = film[:, 64:]
    zm = z * (1.0 + gamma) + beta
    u = (jnp.dot(zm, d1wa_ref[...], preferred_element_type=jnp.float32)
         + d1b_ref[...])
    v = jnp.dot(zm, d1wb_ref[...], preferred_element_type=jnp.float32)
    u_ref[...] = jnp.pad(u, ((0, NP - N), (0, 0)))
    v_ref[...] = jnp.pad(v, ((0, NP - N), (0, 0)))


def _pool_film(agg2a, agg2b, dinv, b2, gateW, gateb, f1W, f1b, f2W, f2b,
               d1Wa, d1Wb, d1b):
    return pl.pallas_call(
        _pool_body,
        out_shape=[jax.ShapeDtypeStruct((NP, 128), jnp.float32),
                   jax.ShapeDtypeStruct((NP, 128), jnp.float32)],
    )(agg2a, agg2b, dinv, b2, gateW, gateb, f1W, f1b, f2W, f2b,
      d1Wa, d1Wb, d1b)


_DEC_BLK = 2000


def _dec_body(ug_ref, vg_ref, w2_ref, b2_ref, out_ref):
    h = jnp.maximum(ug_ref[...] + vg_ref[...], 0.0)
    out_ref[...] = (jnp.dot(h, w2_ref[...], preferred_element_type=jnp.float32)
                    + b2_ref[...])


def _decode_mlp(ug, vg, d2W, d2b):
    return pl.pallas_call(
        _dec_body,
        grid=(E // _DEC_BLK,),
        in_specs=[
            pl.BlockSpec((_DEC_BLK, 128), lambda i: (i, 0)),
            pl.BlockSpec((_DEC_BLK, 128), lambda i: (i, 0)),
            pl.BlockSpec((128, 1), lambda i: (0, 0)),
            pl.BlockSpec((1, 1), lambda i: (0, 0)),
        ],
        out_specs=pl.BlockSpec((_DEC_BLK, 1), lambda i: (i, 0)),
        out_shape=jax.ShapeDtypeStruct((E, 1), jnp.float32),
    )(ug, vg, d2W, d2b)


def kernel(x, edge_index, batch, W1, b1, W2, b2, gateW, gateb,
           f1W, f1b, f2W, f2b, d1W, d1b, d2W, d2b):
    pad = jnp.full((EP - E,), N, jnp.int32)
    srcp = jnp.concatenate([edge_index[0], pad]).reshape(R, G)
    dstp = jnp.concatenate([edge_index[1], pad]).reshape(R, G)

    zerosNP = jnp.zeros((NP, 128), jnp.float32)
    ones128 = jnp.ones((G, 128), jnp.float32)
    dega, degb = _deg_kernel(dstp, zerosNP, ones128)

    hs1a, hs1b, dinv = _mm1(x, W1, dega, degb)
    agg1a, agg1b = _spmm128(hs1a, hs1b, srcp, dstp)

    b1a = b1[:128].reshape(1, 128)
    b1b = b1[128:].reshape(1, 128)
    hs2 = _mm2(agg1a, agg1b, dinv, b1a, b1b, W2[:128], W2[128:])
    agg2a, agg2b = _spmm_pad(hs2, zerosNP, srcp, dstp)

    u, v = _pool_film(agg2a, agg2b, dinv, b2.reshape(1, 64), gateW,
                      gateb.reshape(1, 1), f1W, f1b.reshape(1, 64),
                      f2W, f2b.reshape(1, 128), d1W[:64], d1W[64:],
                      d1b.reshape(1, 128))

    ug, vg = _decode_gather(u, v, srcp, dstp)
    logits = _decode_mlp(ug, vg, d2W, d2b.reshape(1, 1))
    return logits.reshape(E)


# speedup vs baseline: 4.7810x; 1.1395x over previous
import functools

import jax
import jax.numpy as jnp
from jax import lax
from jax.experimental import pallas as pl
from jax.experimental.pallas import tpu as pltpu
from jax.experimental.pallas import tpu_sc as plsc

N = 10000
E = 160000
NC = 2
NS = 16
G = 128
R = 1280
EP = R * G
NP = N + 8

_mesh = lambda: plsc.VectorSubcoreMesh(core_axis_name="c", subcore_axis_name="s")


def _pipelined_rows(n, make_src, rows0, rows1, semA, semB, consume):
    pltpu.async_copy(make_src(0), rows0, semA)

    def body(jj, _):
        j0 = 2 * jj
        j1 = j0 + 1
        pltpu.async_copy(make_src(j1), rows1, semB)
        pltpu.make_async_copy(make_src(j0), rows0, semA).wait()
        consume(j0, rows0)

        @pl.when(jj < n // 2 - 1)
        def _():
            pltpu.async_copy(make_src(j0 + 2), rows0, semA)

        pltpu.make_async_copy(make_src(j1), rows1, semB).wait()
        consume(j1, rows1)
        return 0

    lax.fori_loop(0, n // 2, body, 0)

_DEG_R_SUB = R // (NC * NS)


def _deg_body(dst_hbm, zeros_hbm, ones_hbm, dega_hbm, degb_hbm,
              idx_v, ones_v, acc_sh):
    c = lax.axis_index("c")
    s = lax.axis_index("s")
    w = c * NS + s

    @pl.when(s == 0)
    def _():
        pltpu.sync_copy(zeros_hbm, acc_sh)
    pltpu.sync_copy(ones_hbm, ones_v)
    pltpu.sync_copy(dst_hbm.at[pl.ds(w * _DEG_R_SUB, _DEG_R_SUB)], idx_v)
    plsc.subcore_barrier()

    def body(j, _):
        pltpu.sync_copy(ones_v, acc_sh.at[idx_v.at[j]], add=True)
        return 0

    lax.fori_loop(0, _DEG_R_SUB, body, 0)
    plsc.subcore_barrier()

    @pl.when(jnp.logical_and(c == 0, s == 0))
    def _():
        pltpu.sync_copy(acc_sh, dega_hbm)

    @pl.when(jnp.logical_and(c == 1, s == 0))
    def _():
        pltpu.sync_copy(acc_sh, degb_hbm)


def _deg_kernel(dstp, zeros128, ones128):
    f = pl.kernel(
        _deg_body,
        out_type=[jax.ShapeDtypeStruct((NP, 128), jnp.float32),
                  jax.ShapeDtypeStruct((NP, 128), jnp.float32)],
        mesh=_mesh(),
        scratch_types=[pltpu.VMEM((_DEG_R_SUB, G), jnp.int32),
                       pltpu.VMEM((G, 128), jnp.float32),
                       pltpu.VMEM_SHARED((NP, 128), jnp.float32)],
    )
    return f(dstp, zeros128, ones128)


_SP1_R_SUB = R // NS
_ROWS_PER_SUB = N // 10


_SP1_HALF = _SP1_R_SUB // 2


def _spmm128_body(hs_a, hs_b, src_hbm, dst_hbm, outa_hbm, outb_hbm,
                  src_v, dst_v, rows0, rows1, acc_sh, semA, semB):
    c = lax.axis_index("c")
    s = lax.axis_index("s")

    def run(hs, out_hbm):
        @pl.when(s < 10)
        def _():
            r = pl.ds(s * _ROWS_PER_SUB, _ROWS_PER_SUB)
            pltpu.sync_copy(hs.at[r], acc_sh.at[r])
        plsc.subcore_barrier()

        for h in range(2):
            base = s * _SP1_R_SUB + h * _SP1_HALF
            pltpu.sync_copy(src_hbm.at[pl.ds(base, _SP1_HALF)], src_v)
            pltpu.sync_copy(dst_hbm.at[pl.ds(base, _SP1_HALF)], dst_v)
            _pipelined_rows(
                _SP1_HALF,
                lambda j: hs.at[src_v.at[j]],
                rows0, rows1, semA, semB,
                lambda j, buf: pltpu.sync_copy(
                    buf, acc_sh.at[dst_v.at[j]], add=True),
            )
        plsc.subcore_barrier()

        @pl.when(s < 10)
        def _():
            r = pl.ds(s * _ROWS_PER_SUB, _ROWS_PER_SUB)
            pltpu.sync_copy(acc_sh.at[r], out_hbm.at[r])

    @pl.when(c == 0)
    def _():
        run(hs_a, outa_hbm)

    @pl.when(c == 1)
    def _():
        run(hs_b, outb_hbm)


def _spmm128(hs_a, hs_b, srcp, dstp):
    f = pl.kernel(
        _spmm128_body,
        out_type=[jax.ShapeDtypeStruct((NP, 128), jnp.float32),
                  jax.ShapeDtypeStruct((NP, 128), jnp.float32)],
        mesh=_mesh(),
        scratch_types=[pltpu.VMEM((_SP1_HALF, G), jnp.int32),
                       pltpu.VMEM((_SP1_HALF, G), jnp.int32),
                       pltpu.VMEM((G, 128), jnp.float32),
                       pltpu.VMEM((G, 128), jnp.float32),
                       pltpu.VMEM_SHARED((NP, 128), jnp.float32),
                       pltpu.SemaphoreType.DMA,
                       pltpu.SemaphoreType.DMA],
    )
    return f(hs_a, hs_b, srcp, dstp)


_SP2_R_SUB = R // (NC * NS)


def _spmm_pad_body(hs, zeros_hbm, src_hbm, dst_hbm, outa_hbm, outb_hbm,
                   src_v, dst_v, rows0, rows1, acc_sh, semA, semB):
    c = lax.axis_index("c")
    s = lax.axis_index("s")
    w = c * NS + s

    @pl.when(jnp.logical_and(c == 0, s < 10))
    def _():
        r = pl.ds(s * _ROWS_PER_SUB, _ROWS_PER_SUB)
        pltpu.sync_copy(hs.at[r], acc_sh.at[r])

    @pl.when(jnp.logical_and(c == 1, s < 10))
    def _():
        r = pl.ds(s * _ROWS_PER_SUB, _ROWS_PER_SUB)
        pltpu.sync_copy(zeros_hbm.at[r], acc_sh.at[r])

    pltpu.sync_copy(src_hbm.at[pl.ds(w * _SP2_R_SUB, _SP2_R_SUB)], src_v)
    pltpu.sync_copy(dst_hbm.at[pl.ds(w * _SP2_R_SUB, _SP2_R_SUB)], dst_v)
    plsc.subcore_barrier()

    _pipelined_rows(
        _SP2_R_SUB,
        lambda j: hs.at[src_v.at[j]],
        rows0, rows1, semA, semB,
        lambda j, buf: pltpu.sync_copy(buf, acc_sh.at[dst_v.at[j]], add=True),
    )
    plsc.subcore_barrier()

    @pl.when(jnp.logical_and(c == 0, s < 10))
    def _():
        r = pl.ds(s * _ROWS_PER_SUB, _ROWS_PER_SUB)
        pltpu.sync_copy(acc_sh.at[r], outa_hbm.at[r])

    @pl.when(jnp.logical_and(c == 1, s < 10))
    def _():
        r = pl.ds(s * _ROWS_PER_SUB, _ROWS_PER_SUB)
        pltpu.sync_copy(acc_sh.at[r], outb_hbm.at[r])


def _spmm_pad(hs, zerosNP, srcp, dstp):
    f = pl.kernel(
        _spmm_pad_body,
        out_type=[jax.ShapeDtypeStruct((NP, 128), jnp.float32),
                  jax.ShapeDtypeStruct((NP, 128), jnp.float32)],
        mesh=_mesh(),
        scratch_types=[pltpu.VMEM((_SP2_R_SUB, G), jnp.int32),
                       pltpu.VMEM((_SP2_R_SUB, G), jnp.int32),
                       pltpu.VMEM((G, 128), jnp.float32),
                       pltpu.VMEM((G, 128), jnp.float32),
                       pltpu.VMEM_SHARED((NP, 128), jnp.float32),
                       pltpu.SemaphoreType.DMA,
                       pltpu.SemaphoreType.DMA],
    )
    return f(hs, zerosNP, srcp, dstp)


_GAT_R_SUB = R // (NC * NS)


def _gather_body(u_hbm, v_hbm, src_hbm, dst_hbm, ug_hbm, vg_hbm,
                 idx_v, rows0, rows1, semA, semB):
    c = lax.axis_index("c")
    s = lax.axis_index("s")
    w = c * NS + s

    def run(tab_hbm, ind_hbm, out_hbm):
        pltpu.sync_copy(ind_hbm.at[pl.ds(w * _GAT_R_SUB, _GAT_R_SUB)], idx_v)
        _pipelined_rows(
            _GAT_R_SUB,
            lambda j: tab_hbm.at[idx_v.at[j]],
            rows0, rows1, semA, semB,
            lambda j, buf: pltpu.sync_copy(
                buf, out_hbm.at[pl.ds((w * _GAT_R_SUB + j) * G, G)]),
        )

    run(u_hbm, src_hbm, ug_hbm)
    run(v_hbm, dst_hbm, vg_hbm)


def _decode_gather(u, v, srcp, dstp):
    f = pl.kernel(
        _gather_body,
        out_type=[jax.ShapeDtypeStruct((EP, 128), jnp.float32),
                  jax.ShapeDtypeStruct((EP, 128), jnp.float32)],
        mesh=_mesh(),
        scratch_types=[pltpu.VMEM((_GAT_R_SUB, G), jnp.int32),
                       pltpu.VMEM((G, 128), jnp.float32),
                       pltpu.VMEM((G, 128), jnp.float32),
                       pltpu.SemaphoreType.DMA,
                       pltpu.SemaphoreType.DMA],
    )
    return f(u, v, srcp, dstp)


_ROWS_BLK = 1000


def _mm1_body(x_ref, w_ref, dega_ref, degb_ref, hsa_ref, hsb_ref, dinv_ref):
    deg = dega_ref[:, 0:1] + degb_ref[:, 0:1] + 1.0
    dinv = lax.rsqrt(deg)
    hs = jnp.dot(x_ref[...], w_ref[...], preferred_element_type=jnp.float32)
    hs = hs * dinv
    hsa_ref[...] = hs[:, :128]
    hsb_ref[...] = hs[:, 128:]
    dinv_ref[...] = dinv


def _mm1(x, W1, dega, degb):
    return pl.pallas_call(
        _mm1_body,
        grid=(N // _ROWS_BLK,),
        in_specs=[
            pl.BlockSpec((_ROWS_BLK, 256), lambda i: (i, 0)),
            pl.BlockSpec((256, 256), lambda i: (0, 0)),
            pl.BlockSpec((_ROWS_BLK, 128), lambda i: (i, 0)),
            pl.BlockSpec((_ROWS_BLK, 128), lambda i: (i, 0)),
        ],
        out_specs=[
            pl.BlockSpec((_ROWS_BLK, 128), lambda i: (i, 0)),
            pl.BlockSpec((_ROWS_BLK, 128), lambda i: (i, 0)),
            pl.BlockSpec((_ROWS_BLK, 1), lambda i: (i, 0)),
        ],
        out_shape=[jax.ShapeDtypeStruct((NP, 128), jnp.float32),
                   jax.ShapeDtypeStruct((NP, 128), jnp.float32),
                   jax.ShapeDtypeStruct((N, 1), jnp.float32)],
    )(x, W1, dega, degb)


def _mm2_body(a_ref, b_ref, dinv_ref, b1a_ref, b1b_ref, w2a_ref, w2b_ref,
              hs2_ref):
    dinv = dinv_ref[...]
    ha = jnp.maximum(a_ref[...] * dinv + b1a_ref[...], 0.0)
    hb = jnp.maximum(b_ref[...] * dinv + b1b_ref[...], 0.0)
    zl = (jnp.dot(ha, w2a_ref[...], preferred_element_type=jnp.float32)
          + jnp.dot(hb, w2b_ref[...], preferred_element_type=jnp.float32))
    hs2 = zl * dinv
    hs2_ref[...] = jnp.pad(hs2, ((0, 0), (0, 64)))


def _mm2(agg1a, agg1b, dinv, b1a, b1b, W2a, W2b):
    return pl.pallas_call(
        _mm2_body,
        grid=(N // _ROWS_BLK,),
        in_specs=[
            pl.BlockSpec((_ROWS_BLK, 128), lambda i: (i, 0)),
            pl.BlockSpec((_ROWS_BLK, 128), lambda i: (i, 0)),
            pl.BlockSpec((_ROWS_BLK, 1), lambda i: (i, 0)),
            pl.BlockSpec((1, 128), lambda i: (0, 0)),
            pl.BlockSpec((1, 128), lambda i: (0, 0)),
            pl.BlockSpec((128, 64), lambda i: (0, 0)),
            pl.BlockSpec((128, 64), lambda i: (0, 0)),
        ],
        out_specs=pl.BlockSpec((_ROWS_BLK, 128), lambda i: (i, 0)),
        out_shape=jax.ShapeDtypeStruct((NP, 128), jnp.float32),
    )(agg1a, agg1b, dinv, b1a, b1b, W2a, W2b)


def _pool_body(a_ref, b_ref, dinv_ref, b2_ref, gw_ref, gb_ref,
               f1w_ref, f1b_ref, f2w_ref, f2b_ref, d1wa_ref, d1wb_ref,
               d1b_ref, u_ref, v_ref):
    z = a_ref[0:N, 0:64] + b_ref[0:N, 0:64]
    z = z * dinv_ref[...] + b2_ref[...]
    gate = jnp.dot(z, gw_ref[...], preferred_element_type=jnp.float32) + gb_ref[...]
    gmax = jnp.max(gate, axis=0, keepdims=True)
    e = jnp.exp(gate - gmax)
    den = jnp.sum(e, axis=0, keepdims=True)
    attn = e / den
    g = jnp.sum(attn * z, axis=0, keepdims=True)
    f1 = jnp.maximum(
        jnp.dot(g, f1w_ref[...], preferred_element_type=jnp.float32)
        + f1b_ref[...], 0.0)
    film = jnp.dot(f1, f2w_ref[...], preferred_element_type=jnp.float32) + f2b_ref[...]
    gamma = film[:, :64]
    beta = film[:, 64:]
    zm = z * (1.0 + gamma) + beta
    u = (jnp.dot(zm, d1wa_ref[...], preferred_element_type=jnp.float32)
         + d1b_ref[...])
    v = jnp.dot(zm, d1wb_ref[...], preferred_element_type=jnp.float32)
    u_ref[...] = jnp.pad(u, ((0, NP - N), (0, 0)))
    v_ref[...] = jnp.pad(v, ((0, NP - N), (0, 0)))


def _pool_film(agg2a, agg2b, dinv, b2, gateW, gateb, f1W, f1b, f2W, f2b,
               d1Wa, d1Wb, d1b):
    return pl.pallas_call(
        _pool_body,
        out_shape=[jax.ShapeDtypeStruct((NP, 128), jnp.float32),
                   jax.ShapeDtypeStruct((NP, 128), jnp.float32)],
    )(agg2a, agg2b, dinv, b2, gateW, gateb, f1W, f1b, f2W, f2b,
      d1Wa, d1Wb, d1b)


_DEC_BLK = 2000


def _dec_body(ug_ref, vg_ref, w2_ref, b2_ref, out_ref):
    h = jnp.maximum(ug_ref[...] + vg_ref[...], 0.0)
    out_ref[...] = (jnp.dot(h, w2_ref[...], preferred_element_type=jnp.float32)
                    + b2_ref[...])


def _decode_mlp(ug, vg, d2W, d2b):
    return pl.pallas_call(
        _dec_body,
        grid=(E // _DEC_BLK,),
        in_specs=[
            pl.BlockSpec((_DEC_BLK, 128), lambda i: (i, 0)),
            pl.BlockSpec((_DEC_BLK, 128), lambda i: (i, 0)),
            pl.BlockSpec((128, 1), lambda i: (0, 0)),
            pl.BlockSpec((1, 1), lambda i: (0, 0)),
        ],
        out_specs=pl.BlockSpec((_DEC_BLK, 1), lambda i: (i, 0)),
        out_shape=jax.ShapeDtypeStruct((E, 1), jnp.float32),
    )(ug, vg, d2W, d2b)


def kernel(x, edge_index, batch, W1, b1, W2, b2, gateW, gateb,
           f1W, f1b, f2W, f2b, d1W, d1b, d2W, d2b):
    pad = jnp.full((EP - E,), N, jnp.int32)
    srcp = jnp.concatenate([edge_index[0], pad]).reshape(R, G)
    dstp = jnp.concatenate([edge_index[1], pad]).reshape(R, G)

    zerosNP = jnp.zeros((NP, 128), jnp.float32)
    ones128 = jnp.ones((G, 128), jnp.float32)
    dega, degb = _deg_kernel(dstp, zerosNP, ones128)

    hs1a, hs1b, dinv = _mm1(x, W1, dega, degb)
    agg1a, agg1b = _spmm128(hs1a, hs1b, srcp, dstp)

    b1a = b1[:128].reshape(1, 128)
    b1b = b1[128:].reshape(1, 128)
    hs2 = _mm2(agg1a, agg1b, dinv, b1a, b1b, W2[:128], W2[128:])
    agg2a, agg2b = _spmm_pad(hs2, zerosNP, srcp, dstp)

    u, v = _pool_film(agg2a, agg2b, dinv, b2.reshape(1, 64), gateW,
                      gateb.reshape(1, 1), f1W, f1b.reshape(1, 64),
                      f2W, f2b.reshape(1, 128), d1W[:64], d1W[64:],
                      d1b.reshape(1, 128))

    ug, vg = _decode_gather(u, v, srcp, dstp)
    logits = _decode_mlp(ug, vg, d2W, d2b.reshape(1, 1))
    return logits.reshape(E)
